# Initial kernel scaffold; baseline (speedup 1.0000x reference)
import functools

import jax
import jax.numpy as jnp
from jax import lax
from jax.experimental import pallas as pl
from jax.experimental.pallas import tpu as pltpu
from jax.experimental.pallas import tpu_sc as plsc

_NC = 2
_NS = 16
_CHUNK = 80
_HIGH = lax.Precision.HIGHEST


def _make_seg_kernel(n, e, d, with_count):
    mesh = plsc.VectorSubcoreMesh(core_axis_name="c", subcore_axis_name="s")
    nw = _NC * _NS
    assert e % (nw * _CHUNK) == 0, e
    ew = e // nw
    nchunk = ew // _CHUNK
    assert n % _NS == 0, n
    rps = n // _NS

    outs = [jax.ShapeDtypeStruct((_NC, n, d), jnp.float32)]
    scratch = [
        pltpu.VMEM((_CHUNK,), jnp.int32),
        pltpu.VMEM((_CHUNK,), jnp.int32),
        pltpu.VMEM((_CHUNK, d), jnp.float32),
        pltpu.VMEM_SHARED((n, d), jnp.float32),
        pltpu.SemaphoreType.DMA,
    ]
    if with_count:
        outs.append(jax.ShapeDtypeStruct((_NC, n, 16), jnp.float32))
        scratch += [
            pltpu.VMEM((_CHUNK, 16), jnp.float32),
            pltpu.VMEM_SHARED((n, 16), jnp.float32),
        ]

    def body(refs):
        if with_count:
            (p_hbm, src_hbm, dst_hbm, z64_hbm, z16_hbm,
             acc_out, cnt_out,
             src_v, dst_v, rows_v, acc_sh, sem, ones_v, cnt_sh) = refs
        else:
            (p_hbm, src_hbm, dst_hbm, z64_hbm,
             acc_out,
             src_v, dst_v, rows_v, acc_sh, sem) = refs

        cid = lax.axis_index("c")
        sid = lax.axis_index("s")
        w = sid * _NC + cid
        r0 = sid * rps

        pltpu.sync_copy(z64_hbm.at[pl.ds(r0, rps)], acc_sh.at[pl.ds(r0, rps)])
        if with_count:
            pltpu.sync_copy(z16_hbm.at[pl.ds(r0, rps)],
                            cnt_sh.at[pl.ds(r0, rps)])

            @pl.loop(0, _CHUNK)
            def _(i):
                ones_v[i, :] = jnp.ones((16,), jnp.float32)

        plsc.subcore_barrier()

        base0 = w * ew

        @pl.loop(0, nchunk)
        def _(i):
            base = base0 + i * _CHUNK
            pltpu.sync_copy(src_hbm.at[pl.ds(base, _CHUNK)], src_v)
            pltpu.sync_copy(dst_hbm.at[pl.ds(base, _CHUNK)], dst_v)
            pltpu.async_copy(p_hbm.at[src_v], rows_v, sem).wait()
            pltpu.sync_copy(rows_v, acc_sh.at[dst_v], add=True)
            if with_count:
                pltpu.sync_copy(ones_v, cnt_sh.at[dst_v], add=True)

        plsc.subcore_barrier()

        pltpu.sync_copy(acc_sh.at[pl.ds(r0, rps)],
                        acc_out.at[cid, pl.ds(r0, rps)])
        if with_count:
            pltpu.sync_copy(cnt_sh.at[pl.ds(r0, rps)],
                            cnt_out.at[cid, pl.ds(r0, rps)])

    @functools.partial(pl.kernel, out_type=outs, mesh=mesh,
                       scratch_types=scratch)
    def k(*refs):
        body(refs)

    return k


def _dense(h, wl, wr, b):
    n = h.shape[0]
    fo = wl.shape[1]

    def body(h_ref, wl_ref, wr_ref, b_ref, p_ref, r_ref):
        hh = h_ref[...]
        p_ref[...] = jnp.dot(hh, wl_ref[...], precision=_HIGH,
                             preferred_element_type=jnp.float32)
        r_ref[...] = jnp.dot(hh, wr_ref[...], precision=_HIGH,
                             preferred_element_type=jnp.float32) + b_ref[...]

    return pl.pallas_call(
        body,
        out_shape=[jax.ShapeDtypeStruct((n, fo), jnp.float32),
                   jax.ShapeDtypeStruct((n, fo), jnp.float32)],
    )(h, wl, wr, b.reshape(1, fo))


def _mid(acc, cnt, r_prev, wl, wr, b, want_pre):
    n = r_prev.shape[0]
    fo = wl.shape[1]

    def body(acc_ref, cnt_ref, rp_ref, wl_ref, wr_ref, b_ref, *out_refs):
        cnt_col = cnt_ref[0, :, 0:1] + cnt_ref[1, :, 0:1]
        inv = 1.0 / jnp.maximum(cnt_col, 1.0)
        pre = (acc_ref[0] + acc_ref[1]) * inv + rp_ref[...]
        h = jnp.maximum(pre, 0.0)
        if want_pre:
            pre_ref, p_ref, r_ref = out_refs
            pre_ref[...] = pre
        else:
            p_ref, r_ref = out_refs
        p_ref[...] = jnp.dot(h, wl_ref[...], precision=_HIGH,
                             preferred_element_type=jnp.float32)
        r_ref[...] = jnp.dot(h, wr_ref[...], precision=_HIGH,
                             preferred_element_type=jnp.float32) + b_ref[...]

    d = r_prev.shape[1]
    shapes = [jax.ShapeDtypeStruct((n, fo), jnp.float32),
              jax.ShapeDtypeStruct((n, fo), jnp.float32)]
    if want_pre:
        shapes = [jax.ShapeDtypeStruct((n, d), jnp.float32)] + shapes
    return pl.pallas_call(body, out_shape=shapes)(
        acc, cnt, r_prev, wl, wr, b.reshape(1, fo))


def _final(acc, cnt, r_prev):
    n, d = r_prev.shape

    def body(acc_ref, cnt_ref, rp_ref, o_ref):
        cnt_col = cnt_ref[0, :, 0:1] + cnt_ref[1, :, 0:1]
        inv = 1.0 / jnp.maximum(cnt_col, 1.0)
        o_ref[...] = (acc_ref[0] + acc_ref[1]) * inv + rp_ref[...]

    return pl.pallas_call(
        body, out_shape=jax.ShapeDtypeStruct((n, d), jnp.float32))(
            acc, cnt, r_prev)


def kernel(x, edge_index, W_l0, b0, W_r0, W_l1, b1, W_r1, W_l2, b2, W_r2):
    n = x.shape[0]
    e = edge_index.shape[1]
    d = W_l0.shape[1]

    src = edge_index[0]
    dst = edge_index[1]
    z64 = jnp.zeros((n, d), jnp.float32)
    z16 = jnp.zeros((n, 16), jnp.float32)

    seg_cnt = _make_seg_kernel(n, e, d, with_count=True)
    seg = _make_seg_kernel(n, e, d, with_count=False)

    p0, r0 = _dense(x, W_l0, W_r0, b0)
    acc0, cnt = seg_cnt(p0, src, dst, z64, z16)
    p1, r1 = _mid(acc0, cnt, r0, W_l1, W_r1, b1, want_pre=False)
    acc1 = seg(p1, src, dst, z64)
    feature, p2, r2 = _mid(acc1, cnt, r1, W_l2, W_r2, b2, want_pre=True)
    acc2 = seg(p2, src, dst, z64)
    logits = _final(acc2, cnt, r2)

    return (logits, feature)


# speedup vs baseline: 5.1325x; 5.1325x over previous
import functools

import jax
import jax.numpy as jnp
from jax import lax
from jax.experimental import pallas as pl
from jax.experimental.pallas import tpu as pltpu
from jax.experimental.pallas import tpu_sc as plsc

_NC = 2
_NS = 16
_CHUNK = 80
_HIGH = lax.Precision.HIGHEST


def _make_seg_kernel(n, e, d, with_count):
    mesh = plsc.VectorSubcoreMesh(core_axis_name="c", subcore_axis_name="s")
    nw = _NC * _NS
    assert e % (nw * _CHUNK) == 0, e
    ew = e // nw
    nchunk = ew // _CHUNK
    rps = (n // _NS) // 8 * 8
    rlast = n - (_NS - 1) * rps
    assert rps % 8 == 0 and (_NS - 1) * rps % 8 == 0

    outs = jax.ShapeDtypeStruct((_NC, n, d), jnp.float32)
    scratch = [
        pltpu.VMEM((_CHUNK,), jnp.int32),
        pltpu.VMEM((_CHUNK,), jnp.int32),
        pltpu.VMEM((_CHUNK, d), jnp.float32),
        pltpu.VMEM_SHARED((n, d), jnp.float32),
        pltpu.SemaphoreType.DMA,
    ]
    if with_count:
        outs = (outs, jax.ShapeDtypeStruct((_NC, n, 16), jnp.float32))
        scratch += [
            pltpu.VMEM((_CHUNK, 16), jnp.float32),
            pltpu.VMEM_SHARED((n, 16), jnp.float32),
        ]

    def body(refs):
        if with_count:
            (p_hbm, src_hbm, dst_hbm, z64_hbm, z16_hbm,
             acc_out, cnt_out,
             src_v, dst_v, rows_v, acc_sh, sem, ones_v, cnt_sh) = refs
        else:
            (p_hbm, src_hbm, dst_hbm, z64_hbm,
             acc_out,
             src_v, dst_v, rows_v, acc_sh, sem) = refs

        cid = lax.axis_index("c")
        sid = lax.axis_index("s")
        w = sid * _NC + cid
        r0 = pl.multiple_of(sid * rps, 8)

        def slab_copy(mk_src, mk_dst):
            @pl.when(sid < _NS - 1)
            def _():
                pltpu.sync_copy(mk_src(r0, rps), mk_dst(r0, rps))

            @pl.when(sid == _NS - 1)
            def _():
                pltpu.sync_copy(mk_src((_NS - 1) * rps, rlast),
                                mk_dst((_NS - 1) * rps, rlast))

        slab_copy(lambda o, s: z64_hbm.at[pl.ds(o, s)],
                  lambda o, s: acc_sh.at[pl.ds(o, s)])
        if with_count:
            slab_copy(lambda o, s: z16_hbm.at[pl.ds(o, s)],
                      lambda o, s: cnt_sh.at[pl.ds(o, s)])

            @pl.loop(0, _CHUNK)
            def _(i):
                ones_v[i, :] = jnp.ones((16,), jnp.float32)

        plsc.subcore_barrier()

        base0 = w * ew

        @pl.loop(0, nchunk)
        def _(i):
            base = base0 + i * _CHUNK
            pltpu.sync_copy(src_hbm.at[pl.ds(base, _CHUNK)], src_v)
            pltpu.sync_copy(dst_hbm.at[pl.ds(base, _CHUNK)], dst_v)
            pltpu.async_copy(p_hbm.at[src_v], rows_v, sem).wait()
            pltpu.sync_copy(rows_v, acc_sh.at[dst_v], add=True)
            if with_count:
                pltpu.sync_copy(ones_v, cnt_sh.at[dst_v], add=True)

        plsc.subcore_barrier()

        slab_copy(lambda o, s: acc_sh.at[pl.ds(o, s)],
                  lambda o, s: acc_out.at[cid, pl.ds(o, s)])
        if with_count:
            slab_copy(lambda o, s: cnt_sh.at[pl.ds(o, s)],
                      lambda o, s: cnt_out.at[cid, pl.ds(o, s)])

    @functools.partial(
        pl.kernel, out_type=outs, mesh=mesh, scratch_types=scratch,
        compiler_params=pltpu.CompilerParams(use_tc_tiling_on_sc=False))
    def k(*refs):
        body(refs)

    return k


_ROWBLK = 2000


def _dense(h, wl, wr, b):
    n, fi = h.shape
    fo = wl.shape[1]
    blk = _ROWBLK

    def body(h_ref, wl_ref, wr_ref, b_ref, p_ref, r_ref):
        hh = h_ref[...]
        p_ref[...] = jnp.dot(hh, wl_ref[...], precision=_HIGH,
                             preferred_element_type=jnp.float32)
        r_ref[...] = jnp.dot(hh, wr_ref[...], precision=_HIGH,
                             preferred_element_type=jnp.float32) + b_ref[...]

    return pl.pallas_call(
        body,
        grid=(n // blk,),
        in_specs=[pl.BlockSpec((blk, fi), lambda i: (i, 0)),
                  pl.BlockSpec((fi, fo), lambda i: (0, 0)),
                  pl.BlockSpec((fi, fo), lambda i: (0, 0)),
                  pl.BlockSpec((1, fo), lambda i: (0, 0))],
        out_specs=[pl.BlockSpec((blk, fo), lambda i: (i, 0)),
                   pl.BlockSpec((blk, fo), lambda i: (i, 0))],
        out_shape=[jax.ShapeDtypeStruct((n, fo), jnp.float32),
                   jax.ShapeDtypeStruct((n, fo), jnp.float32)],
    )(h, wl, wr, b.reshape(1, fo))


def _mid(acc, cnt, r_prev, wl, wr, b, want_pre):
    n = r_prev.shape[0]
    fo = wl.shape[1]

    def body(acc_ref, cnt_ref, rp_ref, wl_ref, wr_ref, b_ref, *out_refs):
        cnt_col = cnt_ref[0, :, 0:1] + cnt_ref[1, :, 0:1]
        inv = 1.0 / jnp.maximum(cnt_col, 1.0)
        pre = (acc_ref[0] + acc_ref[1]) * inv + rp_ref[...]
        h = jnp.maximum(pre, 0.0)
        if want_pre:
            pre_ref, p_ref, r_ref = out_refs
            pre_ref[...] = pre
        else:
            p_ref, r_ref = out_refs
        p_ref[...] = jnp.dot(h, wl_ref[...], precision=_HIGH,
                             preferred_element_type=jnp.float32)
        r_ref[...] = jnp.dot(h, wr_ref[...], precision=_HIGH,
                             preferred_element_type=jnp.float32) + b_ref[...]

    d = r_prev.shape[1]
    blk = _ROWBLK
    shapes = [jax.ShapeDtypeStruct((n, fo), jnp.float32),
              jax.ShapeDtypeStruct((n, fo), jnp.float32)]
    out_specs = [pl.BlockSpec((blk, fo), lambda i: (i, 0)),
                 pl.BlockSpec((blk, fo), lambda i: (i, 0))]
    if want_pre:
        shapes = [jax.ShapeDtypeStruct((n, d), jnp.float32)] + shapes
        out_specs = [pl.BlockSpec((blk, d), lambda i: (i, 0))] + out_specs
    return pl.pallas_call(
        body,
        grid=(n // blk,),
        in_specs=[pl.BlockSpec((2, blk, d), lambda i: (0, i, 0)),
                  pl.BlockSpec((2, blk, 16), lambda i: (0, i, 0)),
                  pl.BlockSpec((blk, d), lambda i: (i, 0)),
                  pl.BlockSpec((d, fo), lambda i: (0, 0)),
                  pl.BlockSpec((d, fo), lambda i: (0, 0)),
                  pl.BlockSpec((1, fo), lambda i: (0, 0))],
        out_specs=out_specs,
        out_shape=shapes)(acc, cnt, r_prev, wl, wr, b.reshape(1, fo))


def _final(acc, cnt, r_prev):
    n, d = r_prev.shape

    def body(acc_ref, cnt_ref, rp_ref, o_ref):
        cnt_col = cnt_ref[0, :, 0:1] + cnt_ref[1, :, 0:1]
        inv = 1.0 / jnp.maximum(cnt_col, 1.0)
        o_ref[...] = (acc_ref[0] + acc_ref[1]) * inv + rp_ref[...]

    blk = _ROWBLK
    return pl.pallas_call(
        body,
        grid=(n // blk,),
        in_specs=[pl.BlockSpec((2, blk, d), lambda i: (0, i, 0)),
                  pl.BlockSpec((2, blk, 16), lambda i: (0, i, 0)),
                  pl.BlockSpec((blk, d), lambda i: (i, 0))],
        out_specs=pl.BlockSpec((blk, d), lambda i: (i, 0)),
        out_shape=jax.ShapeDtypeStruct((n, d), jnp.float32))(
            acc, cnt, r_prev)


def kernel(x, edge_index, W_l0, b0, W_r0, W_l1, b1, W_r1, W_l2, b2, W_r2):
    n = x.shape[0]
    e = edge_index.shape[1]
    d = W_l0.shape[1]

    src = edge_index[0]
    dst = edge_index[1]
    z64 = jnp.zeros((n, d), jnp.float32)
    z16 = jnp.zeros((n, 16), jnp.float32)

    seg_cnt = _make_seg_kernel(n, e, d, with_count=True)
    seg = _make_seg_kernel(n, e, d, with_count=False)

    p0, r0 = _dense(x, W_l0, W_r0, b0)
    acc0, cnt = seg_cnt(p0, src, dst, z64, z16)
    p1, r1 = _mid(acc0, cnt, r0, W_l1, W_r1, b1, want_pre=False)
    acc1 = seg(p1, src, dst, z64)
    feature, p2, r2 = _mid(acc1, cnt, r1, W_l2, W_r2, b2, want_pre=True)
    acc2 = seg(p2, src, dst, z64)
    logits = _final(acc2, cnt, r2)

    return (logits, feature)


# speedup vs baseline: 14.0245x; 2.7325x over previous
import functools

import jax
import jax.numpy as jnp
from jax import lax
from jax.experimental import pallas as pl
from jax.experimental.pallas import tpu as pltpu
from jax.experimental.pallas import tpu_sc as plsc

_NC = 2
_NS = 16
_CHUNK = 80
_HIGH = lax.Precision.HIGHEST


_NBUF = 5


def _make_seg_kernel(n, e, d, with_count):
    mesh = plsc.VectorSubcoreMesh(core_axis_name="c", subcore_axis_name="s")
    nw = _NC * _NS
    assert e % (nw * _CHUNK) == 0, e
    ew = e // nw
    nchunk = ew // _CHUNK
    assert nchunk % _NBUF == 0, nchunk
    niter = nchunk // _NBUF
    rps = (n // _NS) // 8 * 8
    rlast = n - (_NS - 1) * rps
    assert rps % 8 == 0 and (_NS - 1) * rps % 8 == 0

    outs = jax.ShapeDtypeStruct((_NC, n, d), jnp.float32)
    scratch = [
        pltpu.VMEM((nchunk, _CHUNK), jnp.int32),
        pltpu.VMEM((nchunk, _CHUNK), jnp.int32),
        [pltpu.VMEM((_CHUNK, d), jnp.float32)] * _NBUF,
        pltpu.VMEM_SHARED((n, d), jnp.float32),
        [pltpu.SemaphoreType.DMA] * _NBUF,
        [pltpu.SemaphoreType.DMA] * _NBUF,
        pltpu.SemaphoreType.DMA,
    ]
    if with_count:
        outs = (outs, jax.ShapeDtypeStruct((_NC, n, 16), jnp.float32))
        scratch += [
            pltpu.VMEM((_CHUNK, 16), jnp.float32),
            pltpu.VMEM_SHARED((n, 16), jnp.float32),
            [pltpu.SemaphoreType.DMA] * _NBUF,
        ]

    def body(refs):
        if with_count:
            (p_hbm, src_hbm, dst_hbm, z64_hbm, z16_hbm,
             acc_out, cnt_out,
             src_v, dst_v, rows_v, acc_sh, gsem, ssem, isem,
             ones_v, cnt_sh, csem) = refs
        else:
            (p_hbm, src_hbm, dst_hbm, z64_hbm,
             acc_out,
             src_v, dst_v, rows_v, acc_sh, gsem, ssem, isem) = refs

        cid = lax.axis_index("c")
        sid = lax.axis_index("s")
        w = sid * _NC + cid
        r0 = pl.multiple_of(sid * rps, 8)

        def slab_copy(mk_src, mk_dst):
            @pl.when(sid < _NS - 1)
            def _():
                pltpu.sync_copy(mk_src(r0, rps), mk_dst(r0, rps))

            @pl.when(sid == _NS - 1)
            def _():
                pltpu.sync_copy(mk_src((_NS - 1) * rps, rlast),
                                mk_dst((_NS - 1) * rps, rlast))

        iload0 = pltpu.async_copy(src_hbm.at[w], src_v, isem)
        slab_copy(lambda o, s: z64_hbm.at[pl.ds(o, s)],
                  lambda o, s: acc_sh.at[pl.ds(o, s)])
        if with_count:
            slab_copy(lambda o, s: z16_hbm.at[pl.ds(o, s)],
                      lambda o, s: cnt_sh.at[pl.ds(o, s)])

            @pl.loop(0, _CHUNK)
            def _(i):
                ones_v[i, :] = jnp.ones((16,), jnp.float32)
        iload0.wait()
        pltpu.async_copy(dst_hbm.at[w], dst_v, isem).wait()

        plsc.subcore_barrier()

        def gather(j, b):
            return pltpu.async_copy(p_hbm.at[src_v.at[j]], rows_v[b], gsem[b])

        def gather_wait(j, b):
            pltpu.make_async_copy(p_hbm.at[src_v.at[j]], rows_v[b],
                                  gsem[b]).wait()

        def scatter(j, b):
            descs = [pltpu.async_copy(rows_v[b], acc_sh.at[dst_v.at[j]],
                                      ssem[b], add=True)]
            if with_count:
                descs.append(pltpu.async_copy(ones_v, cnt_sh.at[dst_v.at[j]],
                                              csem[b], add=True))
            return descs

        for b in range(_NBUF):
            gather(b, b)

        @pl.loop(1, niter)
        def _(g):
            scatters = []
            for b in range(_NBUF):
                jprev = (g - 1) * _NBUF + b
                gather_wait(jprev, b)
                scatters.append(scatter(jprev, b))
            for b in range(_NBUF):
                descs = scatters[b]
                descs[0].wait()
                gather(g * _NBUF + b, b)
                for dsc in descs[1:]:
                    dsc.wait()

        last_scatters = []
        for b in range(_NBUF):
            jlast = (niter - 1) * _NBUF + b
            gather_wait(jlast, b)
            last_scatters.append(scatter(jlast, b))
        for descs in last_scatters:
            for dsc in descs:
                dsc.wait()

        plsc.subcore_barrier()

        slab_copy(lambda o, s: acc_sh.at[pl.ds(o, s)],
                  lambda o, s: acc_out.at[cid, pl.ds(o, s)])
        if with_count:
            slab_copy(lambda o, s: cnt_sh.at[pl.ds(o, s)],
                      lambda o, s: cnt_out.at[cid, pl.ds(o, s)])

    @functools.partial(
        pl.kernel, out_type=outs, mesh=mesh, scratch_types=scratch,
        compiler_params=pltpu.CompilerParams(use_tc_tiling_on_sc=False))
    def k(*refs):
        body(refs)

    return k


_ROWBLK = 2000


def _dense(h, wl, wr, b):
    n, fi = h.shape
    fo = wl.shape[1]
    blk = _ROWBLK

    def body(h_ref, wl_ref, wr_ref, b_ref, p_ref, r_ref):
        hh = h_ref[...]
        p_ref[...] = jnp.dot(hh, wl_ref[...], precision=_HIGH,
                             preferred_element_type=jnp.float32)
        r_ref[...] = jnp.dot(hh, wr_ref[...], precision=_HIGH,
                             preferred_element_type=jnp.float32) + b_ref[...]

    return pl.pallas_call(
        body,
        grid=(n // blk,),
        in_specs=[pl.BlockSpec((blk, fi), lambda i: (i, 0)),
                  pl.BlockSpec((fi, fo), lambda i: (0, 0)),
                  pl.BlockSpec((fi, fo), lambda i: (0, 0)),
                  pl.BlockSpec((1, fo), lambda i: (0, 0))],
        out_specs=[pl.BlockSpec((blk, fo), lambda i: (i, 0)),
                   pl.BlockSpec((blk, fo), lambda i: (i, 0))],
        out_shape=[jax.ShapeDtypeStruct((n, fo), jnp.float32),
                   jax.ShapeDtypeStruct((n, fo), jnp.float32)],
    )(h, wl, wr, b.reshape(1, fo))


def _mid(acc, cnt, r_prev, wl, wr, b, want_pre):
    n = r_prev.shape[0]
    fo = wl.shape[1]

    def body(acc_ref, cnt_ref, rp_ref, wl_ref, wr_ref, b_ref, *out_refs):
        cnt_col = cnt_ref[0, :, 0:1] + cnt_ref[1, :, 0:1]
        inv = 1.0 / jnp.maximum(cnt_col, 1.0)
        pre = (acc_ref[0] + acc_ref[1]) * inv + rp_ref[...]
        h = jnp.maximum(pre, 0.0)
        if want_pre:
            pre_ref, p_ref, r_ref = out_refs
            pre_ref[...] = pre
        else:
            p_ref, r_ref = out_refs
        p_ref[...] = jnp.dot(h, wl_ref[...], precision=_HIGH,
                             preferred_element_type=jnp.float32)
        r_ref[...] = jnp.dot(h, wr_ref[...], precision=_HIGH,
                             preferred_element_type=jnp.float32) + b_ref[...]

    d = r_prev.shape[1]
    blk = _ROWBLK
    shapes = [jax.ShapeDtypeStruct((n, fo), jnp.float32),
              jax.ShapeDtypeStruct((n, fo), jnp.float32)]
    out_specs = [pl.BlockSpec((blk, fo), lambda i: (i, 0)),
                 pl.BlockSpec((blk, fo), lambda i: (i, 0))]
    if want_pre:
        shapes = [jax.ShapeDtypeStruct((n, d), jnp.float32)] + shapes
        out_specs = [pl.BlockSpec((blk, d), lambda i: (i, 0))] + out_specs
    return pl.pallas_call(
        body,
        grid=(n // blk,),
        in_specs=[pl.BlockSpec((2, blk, d), lambda i: (0, i, 0)),
                  pl.BlockSpec((2, blk, 16), lambda i: (0, i, 0)),
                  pl.BlockSpec((blk, d), lambda i: (i, 0)),
                  pl.BlockSpec((d, fo), lambda i: (0, 0)),
                  pl.BlockSpec((d, fo), lambda i: (0, 0)),
                  pl.BlockSpec((1, fo), lambda i: (0, 0))],
        out_specs=out_specs,
        out_shape=shapes)(acc, cnt, r_prev, wl, wr, b.reshape(1, fo))


def _final(acc, cnt, r_prev):
    n, d = r_prev.shape

    def body(acc_ref, cnt_ref, rp_ref, o_ref):
        cnt_col = cnt_ref[0, :, 0:1] + cnt_ref[1, :, 0:1]
        inv = 1.0 / jnp.maximum(cnt_col, 1.0)
        o_ref[...] = (acc_ref[0] + acc_ref[1]) * inv + rp_ref[...]

    blk = _ROWBLK
    return pl.pallas_call(
        body,
        grid=(n // blk,),
        in_specs=[pl.BlockSpec((2, blk, d), lambda i: (0, i, 0)),
                  pl.BlockSpec((2, blk, 16), lambda i: (0, i, 0)),
                  pl.BlockSpec((blk, d), lambda i: (i, 0))],
        out_specs=pl.BlockSpec((blk, d), lambda i: (i, 0)),
        out_shape=jax.ShapeDtypeStruct((n, d), jnp.float32))(
            acc, cnt, r_prev)


def kernel(x, edge_index, W_l0, b0, W_r0, W_l1, b1, W_r1, W_l2, b2, W_r2):
    n = x.shape[0]
    e = edge_index.shape[1]
    d = W_l0.shape[1]

    nw = _NC * _NS
    nchunk = e // (nw * _CHUNK)
    src = edge_index[0].reshape(nw, nchunk, _CHUNK)
    dst = edge_index[1].reshape(nw, nchunk, _CHUNK)
    z64 = jnp.zeros((n, d), jnp.float32)
    z16 = jnp.zeros((n, 16), jnp.float32)

    seg_cnt = _make_seg_kernel(n, e, d, with_count=True)
    seg = _make_seg_kernel(n, e, d, with_count=False)

    p0, r0 = _dense(x, W_l0, W_r0, b0)
    acc0, cnt = seg_cnt(p0, src, dst, z64, z16)
    p1, r1 = _mid(acc0, cnt, r0, W_l1, W_r1, b1, want_pre=False)
    acc1 = seg(p1, src, dst, z64)
    feature, p2, r2 = _mid(acc1, cnt, r1, W_l2, W_r2, b2, want_pre=True)
    acc2 = seg(p2, src, dst, z64)
    logits = _final(acc2, cnt, r2)

    return (logits, feature)


# speedup vs baseline: 14.1679x; 1.0102x over previous
import functools

import jax
import jax.numpy as jnp
from jax import lax
from jax.experimental import pallas as pl
from jax.experimental.pallas import tpu as pltpu
from jax.experimental.pallas import tpu_sc as plsc

_NC = 2
_NS = 16
_CHUNK = 80
_HIGH = lax.Precision.HIGHEST


def _make_seg_kernel(n, e, d, with_count):
    mesh = plsc.VectorSubcoreMesh(core_axis_name="c", subcore_axis_name="s")
    nw = _NC * _NS
    assert e % (nw * _CHUNK) == 0, e
    ew = e // nw
    nchunk = ew // _CHUNK
    nbuf = 5
    assert nchunk % nbuf == 0, nchunk
    niter = nchunk // nbuf
    rps = (n // _NS) // 8 * 8
    rlast = n - (_NS - 1) * rps
    assert rps % 8 == 0 and (_NS - 1) * rps % 8 == 0

    outs = jax.ShapeDtypeStruct((_NC, n, d), jnp.float32)
    scratch = [
        pltpu.VMEM((nchunk, _CHUNK), jnp.int32),
        pltpu.VMEM((nchunk, _CHUNK), jnp.int32),
        [pltpu.VMEM((_CHUNK, d), jnp.float32)] * nbuf,
        pltpu.VMEM_SHARED((n, d), jnp.float32),
        [pltpu.SemaphoreType.DMA] * nbuf,
        [pltpu.SemaphoreType.DMA] * nbuf,
        pltpu.SemaphoreType.DMA,
    ]
    if with_count:
        outs = (outs, jax.ShapeDtypeStruct((_NC, n, 16), jnp.float32))
        scratch += [
            pltpu.VMEM((_CHUNK, 16), jnp.float32),
            pltpu.VMEM_SHARED((n, 16), jnp.float32),
            [pltpu.SemaphoreType.DMA] * nbuf,
        ]

    def body(refs):
        if with_count:
            (p_hbm, src_hbm, dst_hbm, z64_hbm, z16_hbm,
             acc_out, cnt_out,
             src_v, dst_v, rows_v, acc_sh, gsem, ssem, isem,
             ones_v, cnt_sh, csem) = refs
        else:
            (p_hbm, src_hbm, dst_hbm, z64_hbm,
             acc_out,
             src_v, dst_v, rows_v, acc_sh, gsem, ssem, isem) = refs

        cid = lax.axis_index("c")
        sid = lax.axis_index("s")
        w = sid * _NC + cid
        r0 = pl.multiple_of(sid * rps, 8)

        def slab_copy(mk_src, mk_dst):
            @pl.when(sid < _NS - 1)
            def _():
                pltpu.sync_copy(mk_src(r0, rps), mk_dst(r0, rps))

            @pl.when(sid == _NS - 1)
            def _():
                pltpu.sync_copy(mk_src((_NS - 1) * rps, rlast),
                                mk_dst((_NS - 1) * rps, rlast))

        pltpu.async_copy(src_hbm.at[w], src_v, isem).wait()
        dload = pltpu.async_copy(dst_hbm.at[w], dst_v, isem)

        def gather(j, b):
            return pltpu.async_copy(p_hbm.at[src_v.at[j]], rows_v[b], gsem[b])

        def gather_wait(j, b):
            pltpu.make_async_copy(p_hbm.at[src_v.at[j]], rows_v[b],
                                  gsem[b]).wait()

        def scatter(j, b):
            descs = [pltpu.async_copy(rows_v[b], acc_sh.at[dst_v.at[j]],
                                      ssem[b], add=True)]
            if with_count:
                descs.append(pltpu.async_copy(ones_v, cnt_sh.at[dst_v.at[j]],
                                              csem[b], add=True))
            return descs

        for b in range(nbuf):
            gather(b, b)

        slab_copy(lambda o, s: z64_hbm.at[pl.ds(o, s)],
                  lambda o, s: acc_sh.at[pl.ds(o, s)])
        if with_count:
            slab_copy(lambda o, s: z16_hbm.at[pl.ds(o, s)],
                      lambda o, s: cnt_sh.at[pl.ds(o, s)])

            @pl.loop(0, _CHUNK)
            def _(i):
                ones_v[i, :] = jnp.ones((16,), jnp.float32)
        dload.wait()

        plsc.subcore_barrier()

        @pl.loop(1, niter)
        def _(g):
            scatters = []
            for b in range(nbuf):
                jprev = (g - 1) * nbuf + b
                gather_wait(jprev, b)
                scatters.append(scatter(jprev, b))
            for b in range(nbuf):
                descs = scatters[b]
                descs[0].wait()
                gather(g * nbuf + b, b)
                for dsc in descs[1:]:
                    dsc.wait()

        last_scatters = []
        for b in range(nbuf):
            jlast = (niter - 1) * nbuf + b
            gather_wait(jlast, b)
            last_scatters.append(scatter(jlast, b))
        for descs in last_scatters:
            for dsc in descs:
                dsc.wait()

        plsc.subcore_barrier()

        slab_copy(lambda o, s: acc_sh.at[pl.ds(o, s)],
                  lambda o, s: acc_out.at[cid, pl.ds(o, s)])
        if with_count:
            slab_copy(lambda o, s: cnt_sh.at[pl.ds(o, s)],
                      lambda o, s: cnt_out.at[cid, pl.ds(o, s)])

    @functools.partial(
        pl.kernel, out_type=outs, mesh=mesh, scratch_types=scratch,
        compiler_params=pltpu.CompilerParams(use_tc_tiling_on_sc=False))
    def k(*refs):
        body(refs)

    return k


_ROWBLK = 2000


def _dense(h, wl, wr, b):
    n, fi = h.shape
    fo = wl.shape[1]
    blk = _ROWBLK

    def body(h_ref, wl_ref, wr_ref, b_ref, p_ref, r_ref):
        hh = h_ref[...]
        p_ref[...] = jnp.dot(hh, wl_ref[...], precision=_HIGH,
                             preferred_element_type=jnp.float32)
        r_ref[...] = jnp.dot(hh, wr_ref[...], precision=_HIGH,
                             preferred_element_type=jnp.float32) + b_ref[...]

    return pl.pallas_call(
        body,
        grid=(n // blk,),
        in_specs=[pl.BlockSpec((blk, fi), lambda i: (i, 0)),
                  pl.BlockSpec((fi, fo), lambda i: (0, 0)),
                  pl.BlockSpec((fi, fo), lambda i: (0, 0)),
                  pl.BlockSpec((1, fo), lambda i: (0, 0))],
        out_specs=[pl.BlockSpec((blk, fo), lambda i: (i, 0)),
                   pl.BlockSpec((blk, fo), lambda i: (i, 0))],
        out_shape=[jax.ShapeDtypeStruct((n, fo), jnp.float32),
                   jax.ShapeDtypeStruct((n, fo), jnp.float32)],
    )(h, wl, wr, b.reshape(1, fo))


def _mid(acc, cnt, r_prev, wl, wr, b, want_pre):
    n = r_prev.shape[0]
    fo = wl.shape[1]

    def body(acc_ref, cnt_ref, rp_ref, wl_ref, wr_ref, b_ref, *out_refs):
        cnt_col = cnt_ref[0, :, 0:1] + cnt_ref[1, :, 0:1]
        inv = 1.0 / jnp.maximum(cnt_col, 1.0)
        pre = (acc_ref[0] + acc_ref[1]) * inv + rp_ref[...]
        h = jnp.maximum(pre, 0.0)
        if want_pre:
            pre_ref, p_ref, r_ref = out_refs
            pre_ref[...] = pre
        else:
            p_ref, r_ref = out_refs
        p_ref[...] = jnp.dot(h, wl_ref[...], precision=_HIGH,
                             preferred_element_type=jnp.float32)
        r_ref[...] = jnp.dot(h, wr_ref[...], precision=_HIGH,
                             preferred_element_type=jnp.float32) + b_ref[...]

    d = r_prev.shape[1]
    blk = _ROWBLK
    shapes = [jax.ShapeDtypeStruct((n, fo), jnp.float32),
              jax.ShapeDtypeStruct((n, fo), jnp.float32)]
    out_specs = [pl.BlockSpec((blk, fo), lambda i: (i, 0)),
                 pl.BlockSpec((blk, fo), lambda i: (i, 0))]
    if want_pre:
        shapes = [jax.ShapeDtypeStruct((n, d), jnp.float32)] + shapes
        out_specs = [pl.BlockSpec((blk, d), lambda i: (i, 0))] + out_specs
    return pl.pallas_call(
        body,
        grid=(n // blk,),
        in_specs=[pl.BlockSpec((2, blk, d), lambda i: (0, i, 0)),
                  pl.BlockSpec((2, blk, 16), lambda i: (0, i, 0)),
                  pl.BlockSpec((blk, d), lambda i: (i, 0)),
                  pl.BlockSpec((d, fo), lambda i: (0, 0)),
                  pl.BlockSpec((d, fo), lambda i: (0, 0)),
                  pl.BlockSpec((1, fo), lambda i: (0, 0))],
        out_specs=out_specs,
        out_shape=shapes)(acc, cnt, r_prev, wl, wr, b.reshape(1, fo))


def _final(acc, cnt, r_prev):
    n, d = r_prev.shape

    def body(acc_ref, cnt_ref, rp_ref, o_ref):
        cnt_col = cnt_ref[0, :, 0:1] + cnt_ref[1, :, 0:1]
        inv = 1.0 / jnp.maximum(cnt_col, 1.0)
        o_ref[...] = (acc_ref[0] + acc_ref[1]) * inv + rp_ref[...]

    blk = _ROWBLK
    return pl.pallas_call(
        body,
        grid=(n // blk,),
        in_specs=[pl.BlockSpec((2, blk, d), lambda i: (0, i, 0)),
                  pl.BlockSpec((2, blk, 16), lambda i: (0, i, 0)),
                  pl.BlockSpec((blk, d), lambda i: (i, 0))],
        out_specs=pl.BlockSpec((blk, d), lambda i: (i, 0)),
        out_shape=jax.ShapeDtypeStruct((n, d), jnp.float32))(
            acc, cnt, r_prev)


def kernel(x, edge_index, W_l0, b0, W_r0, W_l1, b1, W_r1, W_l2, b2, W_r2):
    n = x.shape[0]
    e = edge_index.shape[1]
    d = W_l0.shape[1]

    nw = _NC * _NS
    nchunk = e // (nw * _CHUNK)
    src = edge_index[0].reshape(nw, nchunk, _CHUNK)
    dst = edge_index[1].reshape(nw, nchunk, _CHUNK)
    z64 = jnp.zeros((n, d), jnp.float32)
    z16 = jnp.zeros((n, 16), jnp.float32)

    seg_cnt = _make_seg_kernel(n, e, d, with_count=True)
    seg = _make_seg_kernel(n, e, d, with_count=False)

    p0, r0 = _dense(x, W_l0, W_r0, b0)
    acc0, cnt = seg_cnt(p0, src, dst, z64, z16)
    p1, r1 = _mid(acc0, cnt, r0, W_l1, W_r1, b1, want_pre=False)
    acc1 = seg(p1, src, dst, z64)
    feature, p2, r2 = _mid(acc1, cnt, r1, W_l2, W_r2, b2, want_pre=True)
    acc2 = seg(p2, src, dst, z64)
    logits = _final(acc2, cnt, r2)

    return (logits, feature)


# speedup vs baseline: 16.1425x; 1.1394x over previous
import functools

import jax
import jax.numpy as jnp
from jax import lax
from jax.experimental import pallas as pl
from jax.experimental.pallas import tpu as pltpu
from jax.experimental.pallas import tpu_sc as plsc

_NC = 2
_NS = 16
_CHUNK = 80
_HIGH = lax.Precision.HIGHEST


def _make_seg_kernel(n, e, d, with_count):
    mesh = plsc.VectorSubcoreMesh(core_axis_name="c", subcore_axis_name="s")
    nw = _NC * _NS
    assert e % (nw * _CHUNK) == 0, e
    ew = e // nw
    nchunk = ew // _CHUNK
    nbuf = 5
    assert nchunk % nbuf == 0, nchunk
    niter = nchunk // nbuf
    rps = (n // _NS) // 8 * 8
    rlast = n - (_NS - 1) * rps
    assert rps % 8 == 0 and (_NS - 1) * rps % 8 == 0

    outs = jax.ShapeDtypeStruct((_NC, n, 128), jnp.float32)
    scratch = [
        pltpu.VMEM((nchunk, _CHUNK), jnp.int32),
        pltpu.VMEM((nchunk, _CHUNK), jnp.int32),
        [pltpu.VMEM((_CHUNK, d), jnp.float32)] * nbuf,
        pltpu.VMEM_SHARED((n, d), jnp.float32),
        [pltpu.SemaphoreType.DMA] * nbuf,
        [pltpu.SemaphoreType.DMA] * nbuf,
        pltpu.SemaphoreType.DMA,
    ]
    if with_count:
        outs = (outs, jax.ShapeDtypeStruct((_NC, n, 16), jnp.float32))
        scratch += [
            pltpu.VMEM((_CHUNK, 16), jnp.float32),
            pltpu.VMEM_SHARED((n, 16), jnp.float32),
            [pltpu.SemaphoreType.DMA] * nbuf,
        ]

    def body(refs):
        if with_count:
            (p_hbm, src_hbm, dst_hbm, z64_hbm, z16_hbm,
             acc_out, cnt_out,
             src_v, dst_v, rows_v, acc_sh, gsem, ssem, isem,
             ones_v, cnt_sh, csem) = refs
        else:
            (p_hbm, src_hbm, dst_hbm, z64_hbm,
             acc_out,
             src_v, dst_v, rows_v, acc_sh, gsem, ssem, isem) = refs

        cid = lax.axis_index("c")
        sid = lax.axis_index("s")
        w = sid * _NC + cid
        r0 = pl.multiple_of(sid * rps, 8)

        def slab_copy(mk_src, mk_dst):
            @pl.when(sid < _NS - 1)
            def _():
                pltpu.sync_copy(mk_src(r0, rps), mk_dst(r0, rps))

            @pl.when(sid == _NS - 1)
            def _():
                pltpu.sync_copy(mk_src((_NS - 1) * rps, rlast),
                                mk_dst((_NS - 1) * rps, rlast))

        pltpu.async_copy(src_hbm.at[w], src_v, isem).wait()
        dload = pltpu.async_copy(dst_hbm.at[w], dst_v, isem)

        def gather(j, b):
            return pltpu.async_copy(p_hbm.at[src_v.at[j]], rows_v[b], gsem[b])

        def gather_wait(j, b):
            pltpu.make_async_copy(p_hbm.at[src_v.at[j]], rows_v[b],
                                  gsem[b]).wait()

        def scatter(j, b):
            descs = [pltpu.async_copy(rows_v[b], acc_sh.at[dst_v.at[j]],
                                      ssem[b], add=True)]
            if with_count:
                descs.append(pltpu.async_copy(ones_v, cnt_sh.at[dst_v.at[j]],
                                              csem[b], add=True))
            return descs

        for b in range(nbuf):
            gather(b, b)

        slab_copy(lambda o, s: z64_hbm.at[pl.ds(o, s)],
                  lambda o, s: acc_sh.at[pl.ds(o, s)])
        if with_count:
            slab_copy(lambda o, s: z16_hbm.at[pl.ds(o, s)],
                      lambda o, s: cnt_sh.at[pl.ds(o, s)])

            @pl.loop(0, _CHUNK)
            def _(i):
                ones_v[i, :] = jnp.ones((16,), jnp.float32)
        dload.wait()

        plsc.subcore_barrier()

        @pl.loop(1, niter)
        def _(g):
            scatters = []
            for b in range(nbuf):
                jprev = (g - 1) * nbuf + b
                gather_wait(jprev, b)
                scatters.append(scatter(jprev, b))
            for b in range(nbuf):
                descs = scatters[b]
                descs[0].wait()
                gather(g * nbuf + b, b)
                for dsc in descs[1:]:
                    dsc.wait()

        last_scatters = []
        for b in range(nbuf):
            jlast = (niter - 1) * nbuf + b
            gather_wait(jlast, b)
            last_scatters.append(scatter(jlast, b))
        for descs in last_scatters:
            for dsc in descs:
                dsc.wait()

        plsc.subcore_barrier()

        slab_copy(lambda o, s: acc_sh.at[pl.ds(o, s)],
                  lambda o, s: acc_out.at[cid, pl.ds(o, s), pl.ds(0, d)])
        if with_count:
            slab_copy(lambda o, s: cnt_sh.at[pl.ds(o, s)],
                      lambda o, s: cnt_out.at[cid, pl.ds(o, s)])

    @functools.partial(
        pl.kernel, out_type=outs, mesh=mesh, scratch_types=scratch,
        compiler_params=pltpu.CompilerParams(use_tc_tiling_on_sc=False))
    def k(*refs):
        body(refs)

    return k


_ROWBLK = 2000


def _dense(h, wlp, wr, b):
    n, fi = h.shape
    fo = wr.shape[1]

    def body(h_ref, wl_ref, wr_ref, b_ref, p_ref, r_ref):
        hh = h_ref[...]
        p_ref[...] = jnp.dot(hh, wl_ref[...], precision=_HIGH,
                             preferred_element_type=jnp.float32)
        r_ref[...] = jnp.dot(hh, wr_ref[...], precision=_HIGH,
                             preferred_element_type=jnp.float32) + b_ref[...]

    blk = _ROWBLK
    return pl.pallas_call(
        body,
        grid=(n // blk,),
        in_specs=[pl.BlockSpec((blk, fi), lambda i: (i, 0)),
                  pl.BlockSpec((fi, 128), lambda i: (0, 0)),
                  pl.BlockSpec((fi, fo), lambda i: (0, 0)),
                  pl.BlockSpec((1, fo), lambda i: (0, 0))],
        out_specs=[pl.BlockSpec((blk, 128), lambda i: (i, 0)),
                   pl.BlockSpec((blk, fo), lambda i: (i, 0))],
        out_shape=[jax.ShapeDtypeStruct((n, 128), jnp.float32),
                   jax.ShapeDtypeStruct((n, fo), jnp.float32)],
    )(h, wlp, wr, b.reshape(1, fo))


def _combine_body(acc_ref, cnt_ref, rp_ref, blk, d):
    cnt_col = cnt_ref[0, :, 0:1] + cnt_ref[1, :, 0:1]
    inv = 1.0 / jnp.maximum(cnt_col, 1.0)
    s = acc_ref[0, :, :d] + acc_ref[1, :, :d]
    return s * inv + rp_ref[...]


def _mid(acc2, cnt, r_prev, wlp, wr, b, want_pre):
    n, d = r_prev.shape
    fo = wr.shape[1]
    blk = _ROWBLK

    def body(acc_ref, cnt_ref, rp_ref, wl_ref, wr_ref, b_ref, *out_refs):
        pre = _combine_body(acc_ref, cnt_ref, rp_ref, blk, d)
        h = jnp.maximum(pre, 0.0)
        if want_pre:
            pre_ref, p_ref, r_ref = out_refs
            pre_ref[...] = pre
        else:
            p_ref, r_ref = out_refs
        p_ref[...] = jnp.dot(h, wl_ref[...], precision=_HIGH,
                             preferred_element_type=jnp.float32)
        r_ref[...] = jnp.dot(h, wr_ref[...], precision=_HIGH,
                             preferred_element_type=jnp.float32) + b_ref[...]

    shapes = [jax.ShapeDtypeStruct((n, 128), jnp.float32),
              jax.ShapeDtypeStruct((n, fo), jnp.float32)]
    out_specs = [pl.BlockSpec((blk, 128), lambda i: (i, 0)),
                 pl.BlockSpec((blk, fo), lambda i: (i, 0))]
    if want_pre:
        shapes = [jax.ShapeDtypeStruct((n, d), jnp.float32)] + shapes
        out_specs = [pl.BlockSpec((blk, d), lambda i: (i, 0))] + out_specs
    return pl.pallas_call(
        body,
        grid=(n // blk,),
        in_specs=[pl.BlockSpec((2, blk, 128), lambda i: (0, i, 0)),
                  pl.BlockSpec((2, blk, 16), lambda i: (0, i, 0)),
                  pl.BlockSpec((blk, d), lambda i: (i, 0)),
                  pl.BlockSpec((d, 128), lambda i: (0, 0)),
                  pl.BlockSpec((d, fo), lambda i: (0, 0)),
                  pl.BlockSpec((1, fo), lambda i: (0, 0))],
        out_specs=out_specs,
        out_shape=shapes)(acc2, cnt, r_prev, wlp, wr, b.reshape(1, fo))


def _final(acc2, cnt, r_prev):
    n, d = r_prev.shape
    blk = _ROWBLK

    def body(acc_ref, cnt_ref, rp_ref, o_ref):
        o_ref[...] = _combine_body(acc_ref, cnt_ref, rp_ref, blk, d)

    return pl.pallas_call(
        body,
        grid=(n // blk,),
        in_specs=[pl.BlockSpec((2, blk, 128), lambda i: (0, i, 0)),
                  pl.BlockSpec((2, blk, 16), lambda i: (0, i, 0)),
                  pl.BlockSpec((blk, d), lambda i: (i, 0))],
        out_specs=pl.BlockSpec((blk, d), lambda i: (i, 0)),
        out_shape=jax.ShapeDtypeStruct((n, d), jnp.float32))(
            acc2, cnt, r_prev)


def kernel(x, edge_index, W_l0, b0, W_r0, W_l1, b1, W_r1, W_l2, b2, W_r2):
    n = x.shape[0]
    e = edge_index.shape[1]
    d = W_l0.shape[1]

    nw = _NC * _NS
    nchunk = e // (nw * _CHUNK)
    src = (edge_index[0] * 2).reshape(nw, nchunk, _CHUNK)
    dst = edge_index[1].reshape(nw, nchunk, _CHUNK)
    z64 = jnp.zeros((n, d), jnp.float32)
    z16 = jnp.zeros((n, 16), jnp.float32)

    def padl(wl):
        return jnp.concatenate([wl, jnp.zeros_like(wl)], axis=1)

    seg_cnt = _make_seg_kernel(n, e, d, with_count=True)
    seg = _make_seg_kernel(n, e, d, with_count=False)

    p0, r0 = _dense(x, padl(W_l0), W_r0, b0)
    acc0, cnt = seg_cnt(p0.reshape(2 * n, d), src, dst, z64, z16)
    p1, r1 = _mid(acc0, cnt, r0, padl(W_l1), W_r1, b1, want_pre=False)
    acc1 = seg(p1.reshape(2 * n, d), src, dst, z64)
    feature, p2, r2 = _mid(acc1, cnt, r1, padl(W_l2), W_r2, b2,
                           want_pre=True)
    acc2 = seg(p2.reshape(2 * n, d), src, dst, z64)
    logits = _final(acc2, cnt, r2)

    return (logits, feature)


# speedup vs baseline: 16.5915x; 1.0278x over previous
import functools

import jax
import jax.numpy as jnp
from jax import lax
from jax.experimental import pallas as pl
from jax.experimental.pallas import tpu as pltpu
from jax.experimental.pallas import tpu_sc as plsc

_NC = 2
_NS = 16
_CHUNK = 80
_HIGH = lax.Precision.HIGHEST


def _make_seg_kernel(n, e, d, with_count):
    mesh = plsc.VectorSubcoreMesh(core_axis_name="c", subcore_axis_name="s")
    nw = _NC * _NS
    assert e % (nw * _CHUNK) == 0, e
    ew = e // nw
    nchunk = ew // _CHUNK
    nbuf = 5
    assert nchunk % nbuf == 0, nchunk
    niter = nchunk // nbuf
    rps = (n // _NS) // 8 * 8
    rlast = n - (_NS - 1) * rps
    assert rps % 8 == 0 and (_NS - 1) * rps % 8 == 0

    outs = jax.ShapeDtypeStruct((_NC, n, 128), jnp.float32)
    scratch = [
        pltpu.VMEM((nchunk, _CHUNK), jnp.int32),
        pltpu.VMEM((nchunk, _CHUNK), jnp.int32),
        [pltpu.VMEM((_CHUNK, d), jnp.float32)] * nbuf,
        pltpu.VMEM_SHARED((n, d), jnp.float32),
        [pltpu.SemaphoreType.DMA] * nbuf,
        [pltpu.SemaphoreType.DMA] * nbuf,
        pltpu.SemaphoreType.DMA,
    ]
    if with_count:
        outs = (outs, jax.ShapeDtypeStruct((_NC, n, 128), jnp.float32))
        scratch += [
            pltpu.VMEM((_CHUNK, 16), jnp.float32),
            pltpu.VMEM_SHARED((n, 16), jnp.float32),
            [pltpu.SemaphoreType.DMA] * nbuf,
        ]

    def body(refs):
        if with_count:
            (p_hbm, src_hbm, dst_hbm, z64_hbm, z16_hbm,
             acc_out, cnt_out,
             src_v, dst_v, rows_v, acc_sh, gsem, ssem, isem,
             ones_v, cnt_sh, csem) = refs
        else:
            (p_hbm, src_hbm, dst_hbm, z64_hbm,
             acc_out,
             src_v, dst_v, rows_v, acc_sh, gsem, ssem, isem) = refs

        cid = lax.axis_index("c")
        sid = lax.axis_index("s")
        w = sid * _NC + cid
        r0 = pl.multiple_of(sid * rps, 8)

        def slab_copy(mk_src, mk_dst):
            @pl.when(sid < _NS - 1)
            def _():
                pltpu.sync_copy(mk_src(r0, rps), mk_dst(r0, rps))

            @pl.when(sid == _NS - 1)
            def _():
                pltpu.sync_copy(mk_src((_NS - 1) * rps, rlast),
                                mk_dst((_NS - 1) * rps, rlast))

        pltpu.async_copy(src_hbm.at[w], src_v, isem).wait()
        dload = pltpu.async_copy(dst_hbm.at[w], dst_v, isem)

        def gather(j, b):
            return pltpu.async_copy(p_hbm.at[src_v.at[j]], rows_v[b], gsem[b])

        def gather_wait(j, b):
            pltpu.make_async_copy(p_hbm.at[src_v.at[j]], rows_v[b],
                                  gsem[b]).wait()

        def scatter(j, b):
            descs = [pltpu.async_copy(rows_v[b], acc_sh.at[dst_v.at[j]],
                                      ssem[b], add=True)]
            if with_count:
                descs.append(pltpu.async_copy(ones_v, cnt_sh.at[dst_v.at[j]],
                                              csem[b], add=True))
            return descs

        for b in range(nbuf):
            gather(b, b)

        slab_copy(lambda o, s: z64_hbm.at[pl.ds(o, s)],
                  lambda o, s: acc_sh.at[pl.ds(o, s)])
        if with_count:
            slab_copy(lambda o, s: z16_hbm.at[pl.ds(o, s)],
                      lambda o, s: cnt_sh.at[pl.ds(o, s)])

            @pl.loop(0, _CHUNK)
            def _(i):
                ones_v[i, :] = jnp.ones((16,), jnp.float32)
        dload.wait()

        plsc.subcore_barrier()

        @pl.loop(1, niter)
        def _(g):
            scatters = []
            for b in range(nbuf):
                jprev = (g - 1) * nbuf + b
                gather_wait(jprev, b)
                scatters.append(scatter(jprev, b))
            for b in range(nbuf):
                descs = scatters[b]
                descs[0].wait()
                gather(g * nbuf + b, b)
                for dsc in descs[1:]:
                    dsc.wait()

        last_scatters = []
        for b in range(nbuf):
            jlast = (niter - 1) * nbuf + b
            gather_wait(jlast, b)
            last_scatters.append(scatter(jlast, b))
        for descs in last_scatters:
            for dsc in descs:
                dsc.wait()

        plsc.subcore_barrier()

        slab_copy(lambda o, s: acc_sh.at[pl.ds(o, s)],
                  lambda o, s: acc_out.at[cid, pl.ds(o, s), pl.ds(0, d)])
        if with_count:
            slab_copy(lambda o, s: cnt_sh.at[pl.ds(o, s)],
                      lambda o, s: cnt_out.at[cid, pl.ds(o, s), pl.ds(0, 16)])

    @functools.partial(
        pl.kernel, out_type=outs, mesh=mesh, scratch_types=scratch,
        compiler_params=pltpu.CompilerParams(use_tc_tiling_on_sc=False))
    def k(*refs):
        body(refs)

    return k


_ROWBLK = 2000


def _dense(h, wlp, wr, b):
    n, fi = h.shape
    fo = wr.shape[1]

    def body(h_ref, wl_ref, wr_ref, b_ref, p_ref, r_ref):
        hh = h_ref[...]
        p_ref[...] = jnp.dot(hh, wl_ref[...], precision=_HIGH,
                             preferred_element_type=jnp.float32)
        r_ref[...] = jnp.dot(hh, wr_ref[...], precision=_HIGH,
                             preferred_element_type=jnp.float32) + b_ref[...]

    blk = _ROWBLK
    return pl.pallas_call(
        body,
        grid=(n // blk,),
        in_specs=[pl.BlockSpec((blk, fi), lambda i: (i, 0)),
                  pl.BlockSpec((fi, 128), lambda i: (0, 0)),
                  pl.BlockSpec((fi, fo), lambda i: (0, 0)),
                  pl.BlockSpec((1, fo), lambda i: (0, 0))],
        out_specs=[pl.BlockSpec((blk, 128), lambda i: (i, 0)),
                   pl.BlockSpec((blk, fo), lambda i: (i, 0))],
        out_shape=[jax.ShapeDtypeStruct((n, 128), jnp.float32),
                   jax.ShapeDtypeStruct((n, fo), jnp.float32)],
    )(h, wlp, wr, b.reshape(1, fo))


def _mid(acc2, cnt_or_inv, r_prev, wlp, wr, b, want_pre):
    n, d = r_prev.shape
    fo = wr.shape[1]
    blk = _ROWBLK
    first = not want_pre

    def body(acc_ref, c_ref, rp_ref, wl_ref, wr_ref, b_ref, *out_refs):
        if first:
            cnt_col = c_ref[0, :, 0:1] + c_ref[1, :, 0:1]
            inv = 1.0 / jnp.maximum(cnt_col, 1.0)
        else:
            inv = c_ref[:, 0:1]
        s = acc_ref[0, :, :d] + acc_ref[1, :, :d]
        pre = s * inv + rp_ref[...]
        h = jnp.maximum(pre, 0.0)
        if want_pre:
            pre_ref, p_ref, r_ref = out_refs
            pre_ref[...] = pre
        else:
            p_ref, r_ref, inv_ref = out_refs
            inv_ref[...] = jnp.broadcast_to(inv, (blk, d))
        p_ref[...] = jnp.dot(h, wl_ref[...], precision=_HIGH,
                             preferred_element_type=jnp.float32)
        r_ref[...] = jnp.dot(h, wr_ref[...], precision=_HIGH,
                             preferred_element_type=jnp.float32) + b_ref[...]

    shapes = [jax.ShapeDtypeStruct((n, 128), jnp.float32),
              jax.ShapeDtypeStruct((n, fo), jnp.float32)]
    out_specs = [pl.BlockSpec((blk, 128), lambda i: (i, 0)),
                 pl.BlockSpec((blk, fo), lambda i: (i, 0))]
    if want_pre:
        shapes = [jax.ShapeDtypeStruct((n, d), jnp.float32)] + shapes
        out_specs = [pl.BlockSpec((blk, d), lambda i: (i, 0))] + out_specs
        c_spec = pl.BlockSpec((blk, d), lambda i: (i, 0))
    else:
        shapes = shapes + [jax.ShapeDtypeStruct((n, d), jnp.float32)]
        out_specs = out_specs + [pl.BlockSpec((blk, d), lambda i: (i, 0))]
        c_spec = pl.BlockSpec((2, blk, 128), lambda i: (0, i, 0))
    return pl.pallas_call(
        body,
        grid=(n // blk,),
        in_specs=[pl.BlockSpec((2, blk, 128), lambda i: (0, i, 0)),
                  c_spec,
                  pl.BlockSpec((blk, d), lambda i: (i, 0)),
                  pl.BlockSpec((d, 128), lambda i: (0, 0)),
                  pl.BlockSpec((d, fo), lambda i: (0, 0)),
                  pl.BlockSpec((1, fo), lambda i: (0, 0))],
        out_specs=out_specs,
        out_shape=shapes)(acc2, cnt_or_inv, r_prev, wlp, wr, b.reshape(1, fo))


def _final(acc2, inv64, r_prev):
    n, d = r_prev.shape
    blk = _ROWBLK

    def body(acc_ref, inv_ref, rp_ref, o_ref):
        s = acc_ref[0, :, :d] + acc_ref[1, :, :d]
        o_ref[...] = s * inv_ref[:, 0:1] + rp_ref[...]

    return pl.pallas_call(
        body,
        grid=(n // blk,),
        in_specs=[pl.BlockSpec((2, blk, 128), lambda i: (0, i, 0)),
                  pl.BlockSpec((blk, d), lambda i: (i, 0)),
                  pl.BlockSpec((blk, d), lambda i: (i, 0))],
        out_specs=pl.BlockSpec((blk, d), lambda i: (i, 0)),
        out_shape=jax.ShapeDtypeStruct((n, d), jnp.float32))(
            acc2, inv64, r_prev)


def kernel(x, edge_index, W_l0, b0, W_r0, W_l1, b1, W_r1, W_l2, b2, W_r2):
    n = x.shape[0]
    e = edge_index.shape[1]
    d = W_l0.shape[1]

    nw = _NC * _NS
    nchunk = e // (nw * _CHUNK)
    src = (edge_index[0] * 2).reshape(nw, nchunk, _CHUNK)
    dst = edge_index[1].reshape(nw, nchunk, _CHUNK)
    z64 = jnp.zeros((n, d), jnp.float32)
    z16 = jnp.zeros((n, 16), jnp.float32)

    def padl(wl):
        return jnp.concatenate([wl, jnp.zeros_like(wl)], axis=1)

    seg_cnt = _make_seg_kernel(n, e, d, with_count=True)
    seg = _make_seg_kernel(n, e, d, with_count=False)

    p0, r0 = _dense(x, padl(W_l0), W_r0, b0)
    acc0, cnt = seg_cnt(p0.reshape(2 * n, d), src, dst, z64, z16)
    p1, r1, inv64 = _mid(acc0, cnt, r0, padl(W_l1), W_r1, b1, want_pre=False)
    acc1 = seg(p1.reshape(2 * n, d), src, dst, z64)
    feature, p2, r2 = _mid(acc1, inv64, r1, padl(W_l2), W_r2, b2,
                           want_pre=True)
    acc2 = seg(p2.reshape(2 * n, d), src, dst, z64)
    logits = _final(acc2, inv64, r2)

    return (logits, feature)
